# baseline (speedup 1.0000x reference)
import functools

import jax
import jax.numpy as jnp
from jax import lax
from jax.experimental import pallas as pl
from jax.experimental.pallas import tpu as pltpu
from jax.experimental.pallas import tpu_sc as plsc

LANES = 16
NSLOTS = 4


@functools.lru_cache(maxsize=None)
def _make_sc_kernel(n_rows, seq, hidden, n_types):
    info = plsc.get_sparse_core_info()
    nc, ns = info.num_cores, info.num_subcores
    nw = nc * ns
    assert n_rows % nw == 0
    rows_per_w = n_rows // nw
    assert seq % rows_per_w == 0
    CH = LANES
    n_chunks = rows_per_w // CH
    nh = hidden // LANES
    assert hidden % LANES == 0
    assert n_chunks % NSLOTS == 0 and n_chunks >= 2 * NSLOTS
    assert n_types == 2 * CH

    mesh = plsc.VectorSubcoreMesh(core_axis_name="c", subcore_axis_name="s")
    chunk_elems = CH * hidden

    def body(tt_hbm, seg_hbm, rowt_hbm, colt_hbm, pos_hbm, out_hbm,
             comb_v, b0, b1, b2, b3, tt_v,
             is0, is1, is2, is3, os0, os1, os2, os3):
        bufs = (b0, b1, b2, b3)
        in_sems = (is0, is1, is2, is3)
        out_sems = (os0, os1, os2, os3)

        cid = lax.axis_index("c")
        sid = lax.axis_index("s")
        wid = sid * nc + cid
        row_base = wid * rows_per_w
        s_base = lax.rem(row_base, seq)

        pltpu.sync_copy(seg_hbm, comb_v)
        pltpu.sync_copy(rowt_hbm.at[pl.ds(0, CH)], b0)
        pltpu.sync_copy(rowt_hbm.at[pl.ds(CH, CH)], b1)
        pltpu.sync_copy(colt_hbm.at[pl.ds(0, CH)], b2)
        pltpu.sync_copy(colt_hbm.at[pl.ds(CH, CH)], b3)
        pltpu.sync_copy(tt_hbm.at[pl.ds(row_base, rows_per_w)], tt_v)

        def combine_row(i, carry):
            base = i * hidden
            hbase = (i + CH) * hidden
            for j in range(nh):
                jo = j * LANES
                jds = pl.ds(jo, LANES)
                lo = pl.ds(base + jo, LANES)
                hi = pl.ds(hbase + jo, LANES)
                comb_v[lo] = comb_v[lo] + b0[i, jds] + b2[i, jds]
                comb_v[hi] = comb_v[hi] + b1[i, jds] + b3[i, jds]
            return carry

        lax.fori_loop(0, CH, combine_row, 0)

        def in_copy(c, k):
            return pltpu.make_async_copy(
                pos_hbm.at[pl.ds(s_base + c * CH, CH)], bufs[k], in_sems[k])

        def out_copy(c, k):
            return pltpu.make_async_copy(
                bufs[k], out_hbm.at[pl.ds(row_base + c * CH, CH)],
                out_sems[k])

        in_copy(0, 0).start()
        in_copy(1, 1).start()

        def step(g, carry):
            for k in range(NSLOTS):
                c = g * NSLOTS + k
                in_copy(c, k).wait()
                ttvec = tt_v[pl.ds(c * CH, CH)]
                tb = [ttvec[r] * hidden for r in range(CH)]
                buf = bufs[k]

                @plsc.parallel_loop(0, nh, unroll=4)
                def jbody(j):
                    jo = j * LANES
                    jds = pl.ds(jo, LANES)
                    for r in range(CH):
                        plsc.addupdate(buf.at[r, jds],
                                       comb_v[pl.ds(tb[r] + jo, LANES)])
                out_copy(c, k).start()

                kp = (k + 2) % NSLOTS
                if k < 2:
                    @pl.when(g >= 1)
                    def _wait():
                        out_copy(c + 2 - NSLOTS, kp).wait()
                    in_copy(c + 2, kp).start()
                else:
                    @pl.when(g < (n_chunks // NSLOTS) - 1)
                    def _pre():
                        out_copy(c + 2 - NSLOTS, kp).wait()
                        in_copy(c + 2, kp).start()
            return carry

        lax.fori_loop(0, n_chunks // NSLOTS, step, 0)

        for k in range(NSLOTS):
            out_copy(n_chunks - NSLOTS + k, k).wait()

    return pl.kernel(
        body,
        out_type=jax.ShapeDtypeStruct((n_rows, hidden), jnp.float32),
        mesh=mesh,
        scratch_types=(
            [pltpu.VMEM((n_types * hidden,), jnp.float32)]
            + [pltpu.VMEM((CH, hidden), jnp.float32)] * NSLOTS
            + [pltpu.VMEM((rows_per_w,), jnp.int32)]
            + [pltpu.SemaphoreType.DMA] * (2 * NSLOTS)
        ),
    )


def kernel(input_ids, token_type_ids, seg_table, pos_table, row_table,
           col_table):
    batch, seq = token_type_ids.shape
    n_types, hidden = seg_table.shape
    tt = token_type_ids.astype(jnp.int32).reshape(-1)
    sc = _make_sc_kernel(batch * seq, seq, hidden, n_types)
    out = sc(tt, seg_table.reshape(-1), row_table, col_table, pos_table)
    return out.reshape(batch, seq, hidden)


# speedup vs baseline: 1.0788x; 1.0788x over previous
import functools

import jax
import jax.numpy as jnp
from jax import lax
from jax.experimental import pallas as pl
from jax.experimental.pallas import tpu as pltpu
from jax.experimental.pallas import tpu_sc as plsc

LANES = 16
NSLOTS = 4


@functools.lru_cache(maxsize=None)
def _make_sc_kernel(n_rows, seq, hidden, n_types):
    info = plsc.get_sparse_core_info()
    nc, ns = info.num_cores, info.num_subcores
    nw = nc * ns
    assert n_rows % nw == 0
    rows_per_w = n_rows // nw
    assert seq % rows_per_w == 0
    CH = 2 * LANES
    n_chunks = rows_per_w // CH
    nh = hidden // LANES
    assert hidden % LANES == 0
    assert n_chunks % NSLOTS == 0 and n_chunks >= 2 * NSLOTS
    assert n_types == CH

    mesh = plsc.VectorSubcoreMesh(core_axis_name="c", subcore_axis_name="s")
    chunk_elems = CH * hidden

    def body(tt_hbm, seg_hbm, rowt_hbm, colt_hbm, pos_hbm, out_hbm,
             comb_v, b0, b1, b2, b3, tt_v,
             is0, is1, is2, is3, os0, os1, os2, os3):
        bufs = (b0, b1, b2, b3)
        in_sems = (is0, is1, is2, is3)
        out_sems = (os0, os1, os2, os3)

        cid = lax.axis_index("c")
        sid = lax.axis_index("s")
        wid = sid * nc + cid
        row_base = wid * rows_per_w
        s_base = lax.rem(row_base, seq)

        pltpu.sync_copy(seg_hbm, comb_v)
        pltpu.sync_copy(rowt_hbm, b0)
        pltpu.sync_copy(colt_hbm, b1)
        pltpu.sync_copy(tt_hbm.at[pl.ds(row_base, rows_per_w)], tt_v)

        def combine_row(i, carry):
            base = i * hidden
            for j in range(nh):
                jo = j * LANES
                jds = pl.ds(jo, LANES)
                lo = pl.ds(base + jo, LANES)
                comb_v[lo] = comb_v[lo] + b0[i, jds] + b1[i, jds]
            return carry

        lax.fori_loop(0, CH, combine_row, 0)

        def in_copy(c, k):
            return pltpu.make_async_copy(
                pos_hbm.at[pl.ds(s_base + c * CH, CH)], bufs[k], in_sems[k])

        def out_copy(c, k):
            return pltpu.make_async_copy(
                bufs[k], out_hbm.at[pl.ds(row_base + c * CH, CH)],
                out_sems[k])

        in_copy(0, 0).start()
        in_copy(1, 1).start()

        def step(g, carry):
            for k in range(NSLOTS):
                c = g * NSLOTS + k
                in_copy(c, k).wait()
                ttvec0 = tt_v[pl.ds(c * CH, LANES)]
                ttvec1 = tt_v[pl.ds(c * CH + LANES, LANES)]
                tb = ([ttvec0[r] * hidden for r in range(LANES)]
                      + [ttvec1[r] * hidden for r in range(LANES)])
                buf = bufs[k]

                @plsc.parallel_loop(0, nh, unroll=2)
                def jbody(j):
                    jo = j * LANES
                    jds = pl.ds(jo, LANES)
                    for r in range(CH):
                        plsc.addupdate(buf.at[r, jds],
                                       comb_v[pl.ds(tb[r] + jo, LANES)])
                out_copy(c, k).start()

                kp = (k + 2) % NSLOTS
                if k < 2:
                    @pl.when(g >= 1)
                    def _wait():
                        out_copy(c + 2 - NSLOTS, kp).wait()
                    in_copy(c + 2, kp).start()
                else:
                    @pl.when(g < (n_chunks // NSLOTS) - 1)
                    def _pre():
                        out_copy(c + 2 - NSLOTS, kp).wait()
                        in_copy(c + 2, kp).start()
            return carry

        lax.fori_loop(0, n_chunks // NSLOTS, step, 0)

        for k in range(NSLOTS):
            out_copy(n_chunks - NSLOTS + k, k).wait()

    return pl.kernel(
        body,
        out_type=jax.ShapeDtypeStruct((n_rows, hidden), jnp.float32),
        mesh=mesh,
        scratch_types=(
            [pltpu.VMEM((n_types * hidden,), jnp.float32)]
            + [pltpu.VMEM((CH, hidden), jnp.float32)] * NSLOTS
            + [pltpu.VMEM((rows_per_w,), jnp.int32)]
            + [pltpu.SemaphoreType.DMA] * (2 * NSLOTS)
        ),
    )


def kernel(input_ids, token_type_ids, seg_table, pos_table, row_table,
           col_table):
    batch, seq = token_type_ids.shape
    n_types, hidden = seg_table.shape
    tt = token_type_ids.astype(jnp.int32).reshape(-1)
    sc = _make_sc_kernel(batch * seq, seq, hidden, n_types)
    out = sc(tt, seg_table.reshape(-1), row_table, col_table, pos_table)
    return out.reshape(batch, seq, hidden)


# speedup vs baseline: 1.1114x; 1.0302x over previous
import functools

import jax
import jax.numpy as jnp
from jax import lax
from jax.experimental import pallas as pl
from jax.experimental.pallas import tpu as pltpu
from jax.experimental.pallas import tpu_sc as plsc

LANES = 16


@functools.lru_cache(maxsize=None)
def _make_sc_kernel(batch, seq, hidden, n_types):
    info = plsc.get_sparse_core_info()
    nc, ns = info.num_cores, info.num_subcores
    nw = nc * ns
    assert seq % nw == 0
    s_per_w = seq // nw
    CH = LANES
    n_chunks = s_per_w // CH
    nh = hidden // LANES
    assert hidden % LANES == 0
    assert n_chunks >= 2
    assert n_types == 2 * CH
    assert batch == 4

    mesh = plsc.VectorSubcoreMesh(core_axis_name="c", subcore_axis_name="s")

    def body(tt_hbm, seg_hbm, rowt_hbm, colt_hbm, pos_hbm, out_hbm,
             comb_v, pb0, pb1, ob0, ob1, ob2, ob3, tt_v,
             ip0, ip1, os0, os1, os2, os3):
        pbufs = (pb0, pb1)
        obufs = (ob0, ob1, ob2, ob3)
        in_sems = (ip0, ip1)
        out_sems = (os0, os1, os2, os3)

        cid = lax.axis_index("c")
        sid = lax.axis_index("s")
        wid = sid * nc + cid
        s_base = wid * s_per_w

        pltpu.sync_copy(seg_hbm, comb_v)
        pltpu.sync_copy(rowt_hbm.at[pl.ds(0, CH)], ob0)
        pltpu.sync_copy(rowt_hbm.at[pl.ds(CH, CH)], ob1)
        pltpu.sync_copy(colt_hbm.at[pl.ds(0, CH)], ob2)
        pltpu.sync_copy(colt_hbm.at[pl.ds(CH, CH)], ob3)
        for b in range(batch):
            pltpu.sync_copy(
                tt_hbm.at[pl.ds(b * seq + s_base, s_per_w)],
                tt_v.at[pl.ds(b * s_per_w, s_per_w)])

        def combine_row(i, carry):
            base = i * hidden
            hbase = (i + CH) * hidden
            for j in range(nh):
                jo = j * LANES
                jds = pl.ds(jo, LANES)
                lo = pl.ds(base + jo, LANES)
                hi = pl.ds(hbase + jo, LANES)
                comb_v[lo] = comb_v[lo] + ob0[i, jds] + ob2[i, jds]
                comb_v[hi] = comb_v[hi] + ob1[i, jds] + ob3[i, jds]
            return carry

        lax.fori_loop(0, CH, combine_row, 0)

        def in_copy(c, k):
            return pltpu.make_async_copy(
                pos_hbm.at[pl.ds(s_base + c * CH, CH)], pbufs[k],
                in_sems[k])

        def out_copy(c, b):
            return pltpu.make_async_copy(
                obufs[b],
                out_hbm.at[pl.ds(b * seq + s_base + c * CH, CH)],
                out_sems[b])

        in_copy(0, 0).start()

        def step(g, carry):
            for kk in range(2):
                c = g * 2 + kk
                in_copy(c, kk).wait()

                @pl.when(c + 1 < n_chunks)
                def _pre():
                    in_copy(c + 1, 1 - kk).start()

                pb = pbufs[kk]
                for b in range(batch):
                    @pl.when(c >= 1)
                    def _wo():
                        out_copy(c - 1, b).wait()
                    ttvec = tt_v[pl.ds(b * s_per_w + c * CH, CH)]
                    tb = [ttvec[r] * hidden for r in range(CH)]
                    ob = obufs[b]

                    @plsc.parallel_loop(0, nh, unroll=2)
                    def jbody(j):
                        jo = j * LANES
                        jds = pl.ds(jo, LANES)
                        for r in range(CH):
                            ob[r, jds] = (pb[r, jds]
                                          + comb_v[pl.ds(tb[r] + jo, LANES)])

                    out_copy(c, b).start()
            return carry

        lax.fori_loop(0, n_chunks // 2, step, 0)

        for b in range(batch):
            out_copy(n_chunks - 1, b).wait()

    return pl.kernel(
        body,
        out_type=jax.ShapeDtypeStruct((batch * seq, hidden), jnp.float32),
        mesh=mesh,
        scratch_types=(
            [pltpu.VMEM((n_types * hidden,), jnp.float32)]
            + [pltpu.VMEM((CH, hidden), jnp.float32)] * 6
            + [pltpu.VMEM((batch * s_per_w,), jnp.int32)]
            + [pltpu.SemaphoreType.DMA] * 6
        ),
    )


def kernel(input_ids, token_type_ids, seg_table, pos_table, row_table,
           col_table):
    batch, seq = token_type_ids.shape
    n_types, hidden = seg_table.shape
    tt = token_type_ids.astype(jnp.int32).reshape(-1)
    sc = _make_sc_kernel(batch, seq, hidden, n_types)
    out = sc(tt, seg_table.reshape(-1), row_table, col_table, pos_table)
    return out.reshape(batch, seq, hidden)
